# Initial kernel scaffold; baseline (speedup 1.0000x reference)
import jax
import jax.numpy as jnp
from jax.experimental import pallas as pl


def kernel(x, W_ih, W_hh, b_hh, W_out, b_out, s_hh, s_b_hh, s_out, s_b_out):
    raise NotImplementedError("write your pallas kernel here")


# speedup vs baseline: 13.7154x; 13.7154x over previous
import functools

import jax
import jax.numpy as jnp
from jax import lax
from jax.experimental import pallas as pl
from jax.experimental.pallas import tpu as pltpu
from jax.experimental.pallas import tpu_sc as plsc

_SPARSITY = 0.05


_NW = 32
_CHUNK = 4096
_NVEC = _CHUNK // 16


def _keyify(v):
    ui = lax.bitcast_convert_type(v, jnp.int32)
    sign = ui >> 31
    flip = lax.bitcast_convert_type(sign, jnp.uint32) | jnp.uint32(0x80000000)
    return lax.bitcast_convert_type(v, jnp.uint32) ^ flip


def _make_sc_hist(bins, shift, filt_shift):
    has_filter = filt_shift is not None
    mesh = plsc.VectorSubcoreMesh(core_axis_name="c", subcore_axis_name="s")

    def body(*refs):
        if has_filter:
            (shh, sout, sbhh, sbout, params, out,
             buf, hist, pref_v) = refs
        else:
            (shh, sout, sbhh, sbout, out, buf, hist) = refs
        wid = lax.axis_index("s") * 2 + lax.axis_index("c")
        lane = lax.iota(jnp.int32, 16)

        def clr(i, c):
            hist[pl.ds(i * 16, 16)] = jnp.zeros((16,), jnp.int32)
            return c
        lax.fori_loop(0, bins, clr, 0)

        if has_filter:
            pltpu.sync_copy(params.at[pl.ds(0, 16)], pref_v)
            prefix = pref_v[...].astype(jnp.uint32)

        def vec_body(i, c):
            v = buf[pl.ds(i * 16, 16)]
            key = _keyify(v)
            field = ((key >> shift) & jnp.uint32(bins - 1)).astype(jnp.int32)
            idx = lane * bins + field
            if has_filter:
                ok = (key >> filt_shift) == prefix
                val = jnp.where(ok, 1, 0).astype(jnp.int32)
            else:
                val = jnp.full((16,), 1, jnp.int32)
            plsc.addupdate_scatter(hist, [idx], val)
            return c

        def chunk_loop(mat):
            def cb(j, c):
                off = (wid * 32 + j) * _CHUNK
                pltpu.sync_copy(mat.at[pl.ds(off, _CHUNK)], buf)
                lax.fori_loop(0, _NVEC, vec_body, 0)
                return c
            lax.fori_loop(0, 32, cb, 0)

        chunk_loop(shh)
        chunk_loop(sout)

        @pl.when(wid == 0)
        def _():
            pltpu.sync_copy(sbhh.at[pl.ds(0, 2048)], buf.at[pl.ds(0, 2048)])
            lax.fori_loop(0, 128, vec_body, 0)

        @pl.when(wid == 1)
        def _():
            pltpu.sync_copy(sbout.at[pl.ds(0, 2048)], buf.at[pl.ds(0, 2048)])
            lax.fori_loop(0, 128, vec_body, 0)

        pltpu.sync_copy(hist, out.at[wid])

    scratch = [
        pltpu.VMEM((_CHUNK,), jnp.float32),
        pltpu.VMEM((16 * bins,), jnp.int32),
    ]
    if has_filter:
        scratch.append(pltpu.VMEM((16,), jnp.int32))

    return pl.kernel(
        body,
        mesh=mesh,
        out_type=jax.ShapeDtypeStruct((_NW, 16 * bins), jnp.int32),
        scratch_types=scratch,
        compiler_params=pltpu.CompilerParams(needs_layout_passes=False),
    )


def _suffix_counts(hist_rows, bins):
    hist = jnp.sum(hist_rows.astype(jnp.float32), axis=0, keepdims=True)
    row = lax.broadcasted_iota(jnp.int32, (bins, bins), 0)
    col = lax.broadcasted_iota(jnp.int32, (bins, bins), 1)
    m = jnp.where(row >= col, 1.0, 0.0).astype(jnp.float32)
    counts_ge = lax.dot_general(hist, m, (((1,), (0,)), ((), ())),
                                precision=lax.Precision.HIGHEST,
                                preferred_element_type=jnp.float32)
    return counts_ge, hist


def _pick(counts_ge, hist, kcur, bins):
    ok = counts_ge >= kcur
    iota_b = lax.broadcasted_iota(jnp.int32, (1, bins), 1)
    bstar = jnp.max(jnp.where(ok, iota_b, -1))
    counts_gt = counts_ge - hist
    gt_at = jnp.sum(jnp.where(iota_b == bstar, counts_gt, 0.0))
    knext = kcur - gt_at
    return bstar, knext


def _make_sel(bins, level, k_total):
    def body(*refs):
        if level == 1:
            hist_ref, out_ref = refs
            kcur = jnp.float32(k_total)
            prev_prefix = jnp.int32(0)
        else:
            hist_ref, aux_ref, out_ref = refs
            kcur = aux_ref[1, 0].astype(jnp.float32)
            prev_prefix = aux_ref[0, 0]
        counts_ge, hist = _suffix_counts(hist_ref[...], bins)
        bstar, knext = _pick(counts_ge, hist, kcur, bins)
        prefix_next = prev_prefix * bins + bstar
        if level < 3:
            rowi = lax.broadcasted_iota(jnp.int32, (8, 128), 0)
            out_ref[...] = jnp.where(rowi == 0, prefix_next,
                                     knext.astype(jnp.int32))
        else:
            key = prefix_next.astype(jnp.uint32)
            keyi = lax.bitcast_convert_type(key, jnp.int32)
            bits = jnp.where(keyi < 0, key ^ jnp.uint32(0x80000000), ~key)
            thr = lax.bitcast_convert_type(bits, jnp.float32)
            out_ref[...] = jnp.full((8, 128), thr, jnp.float32)

    out_dtype = jnp.int32 if level < 3 else jnp.float32
    n_in = 1 if level == 1 else 2
    return pl.pallas_call(
        body,
        out_shape=jax.ShapeDtypeStruct((8, 128), out_dtype),
        in_specs=[pl.BlockSpec((512, bins), lambda: (0, 0))] +
                 ([pl.BlockSpec((8, 128), lambda: (0, 0))] if n_in == 2 else []),
        out_specs=pl.BlockSpec((8, 128), lambda: (0, 0)),
    )


_BM = 256
_BX = 512


def _mask_body(thr_ref, whh_ref, shh_ref, wout_ref, sout_ref,
               bhh_ref, sbhh_ref, bout_ref, sbout_ref,
               wmhh_ref, wmout_ref, bmhh_ref, bmout_ref):
    thr = thr_ref[0, 0]
    wmhh_ref[...] = jnp.where(shh_ref[...] >= thr, whh_ref[...], 0.0
                              ).astype(jnp.bfloat16)
    wmout_ref[...] = jnp.where(sout_ref[...] >= thr, wout_ref[...], 0.0
                               ).astype(jnp.bfloat16)
    bmhh_ref[...] = jnp.where(sbhh_ref[...] >= thr, bhh_ref[...], 0.0)
    bmout_ref[...] = jnp.where(sbout_ref[...] >= thr, bout_ref[...], 0.0)


def _apply_masks(thr, W_hh, s_hh, W_out, s_out, b_hh, s_b_hh, b_out, s_b_out):
    d_h, _ = W_hh.shape
    grid = (d_h // _BM,)
    row_spec = pl.BlockSpec((_BM, W_hh.shape[1]), lambda i: (i, 0))
    vec_spec = pl.BlockSpec((1, W_hh.shape[1]), lambda i: (0, 0))
    thr_spec = pl.BlockSpec((8, 128), lambda i: (0, 0))
    return pl.pallas_call(
        _mask_body,
        grid=grid,
        in_specs=[thr_spec, row_spec, row_spec, row_spec, row_spec,
                  vec_spec, vec_spec, vec_spec, vec_spec],
        out_specs=[row_spec, row_spec, vec_spec, vec_spec],
        out_shape=[
            jax.ShapeDtypeStruct(W_hh.shape, jnp.bfloat16),
            jax.ShapeDtypeStruct(W_out.shape, jnp.bfloat16),
            jax.ShapeDtypeStruct((1, d_h), jnp.float32),
            jax.ShapeDtypeStruct((1, d_h), jnp.float32),
        ],
    )(thr, W_hh, s_hh, W_out, s_out, b_hh, s_b_hh, b_out, s_b_out)


def _mm_body(x_ref, wih_ref, wmhh_ref, bmhh_ref, wmout_ref, bmout_ref,
             out_ref):
    nt = (((1,), (1,)), ((), ()))
    h = jnp.tanh(lax.dot_general(x_ref[...], wih_ref[...], nt,
                                 preferred_element_type=jnp.float32))
    h2 = jnp.tanh(lax.dot_general(h.astype(jnp.bfloat16), wmhh_ref[...], nt,
                                  preferred_element_type=jnp.float32)
                  + bmhh_ref[...])
    out_ref[...] = lax.dot_general(h2.astype(jnp.bfloat16), wmout_ref[...], nt,
                                   preferred_element_type=jnp.float32
                                   ) + bmout_ref[...]


def _forward(x_bf, wih_bf, wmhh, bmhh, wmout, bmout):
    n_tok, d_in = x_bf.shape
    d_h = wih_bf.shape[0]
    grid = (n_tok // _BX,)
    full = lambda s: pl.BlockSpec(s, lambda i: (0, 0))
    return pl.pallas_call(
        _mm_body,
        grid=grid,
        in_specs=[pl.BlockSpec((_BX, d_in), lambda i: (i, 0)),
                  full(wih_bf.shape), full(wmhh.shape), full((1, d_h)),
                  full(wmout.shape), full((1, d_h))],
        out_specs=pl.BlockSpec((_BX, d_h), lambda i: (i, 0)),
        out_shape=jax.ShapeDtypeStruct((n_tok, d_h), jnp.float32),
        compiler_params=pltpu.CompilerParams(
            vmem_limit_bytes=100 * 1024 * 1024),
    )(x_bf, wih_bf, wmhh, bmhh, wmout, bmout)


def kernel(x, W_ih, W_hh, b_hh, W_out, b_out, s_hh, s_b_hh, s_out, s_b_out):
    total = s_hh.size + s_b_hh.size + s_out.size + s_b_out.size
    k = max(1, int(_SPARSITY * total))

    shh_flat = s_hh.reshape(-1)
    sout_flat = s_out.reshape(-1)

    sc1 = _make_sc_hist(2048, 21, None)
    h1 = sc1(shh_flat, sout_flat, s_b_hh, s_b_out)
    p1 = _make_sel(2048, 1, k)(h1.reshape(512, 2048))

    sc2 = _make_sc_hist(2048, 10, 21)
    h2 = sc2(shh_flat, sout_flat, s_b_hh, s_b_out, p1.reshape(-1))
    p2 = _make_sel(2048, 2, k)(h2.reshape(512, 2048), p1)

    sc3 = _make_sc_hist(1024, 0, 10)
    h3 = sc3(shh_flat, sout_flat, s_b_hh, s_b_out, p2.reshape(-1))
    thr = _make_sel(1024, 3, k)(h3.reshape(512, 1024), p2)

    wmhh, wmout, bmhh, bmout = _apply_masks(
        thr, W_hh, s_hh, W_out, s_out,
        b_hh.reshape(1, -1), s_b_hh.reshape(1, -1),
        b_out.reshape(1, -1), s_b_out.reshape(1, -1))

    return _forward(x.astype(jnp.bfloat16), W_ih.astype(jnp.bfloat16),
                    wmhh, bmhh, wmout, bmout)


# speedup vs baseline: 15.3414x; 1.1186x over previous
import functools

import jax
import jax.numpy as jnp
from jax import lax
from jax.experimental import pallas as pl
from jax.experimental.pallas import tpu as pltpu
from jax.experimental.pallas import tpu_sc as plsc

_SPARSITY = 0.05


_NW = 32
_CHUNK = 4096
_NVEC = _CHUNK // 16


def _keyify(v):
    ui = lax.bitcast_convert_type(v, jnp.int32)
    sign = ui >> 31
    flip = lax.bitcast_convert_type(sign, jnp.uint32) | jnp.uint32(0x80000000)
    return lax.bitcast_convert_type(v, jnp.uint32) ^ flip


_UNROLL = 8


def _make_sc_hist(bins, shift, filt_shift):
    has_filter = filt_shift is not None
    mesh = plsc.VectorSubcoreMesh(core_axis_name="c", subcore_axis_name="s")

    def body(*refs):
        if has_filter:
            (shh, sout, sbhh, sbout, params, out,
             buf, hist, sems, prow) = refs
        else:
            (shh, sout, sbhh, sbout, out, buf, hist, sems) = refs
        wid = lax.axis_index("s") * 2 + lax.axis_index("c")
        lane = lax.iota(jnp.int32, 16)

        def clr(i, c):
            z = jnp.zeros((16,), jnp.int32)
            for l in range(16):
                hist[l, pl.ds(i * 16, 16)] = z
            return c
        lax.fori_loop(0, bins // 16, clr, 0)

        if has_filter:
            pltpu.sync_copy(params.at[0], prow)
            prefix = prow[pl.ds(0, 16)].astype(jnp.uint32)

        def process(bslot, nvec8):
            def vb(i, c):
                for u in range(_UNROLL):
                    v = buf[bslot, pl.ds((i * _UNROLL + u) * 16, 16)]
                    key = _keyify(v)
                    field = ((key >> shift) & jnp.uint32(bins - 1)
                             ).astype(jnp.int32)
                    if has_filter:
                        ok = (key >> filt_shift) == prefix
                        val = jnp.where(ok, 1, 0).astype(jnp.int32)
                    else:
                        val = jnp.full((16,), 1, jnp.int32)
                    plsc.addupdate_scatter(hist, [lane, field], val)
                return c
            lax.fori_loop(0, nvec8, vb, 0)

        def chunk_loop(mat):
            first = mat.at[pl.ds(wid * 32 * _CHUNK, _CHUNK)]
            pltpu.make_async_copy(first, buf.at[0], sems.at[0]).start()

            def cb(j, c):
                nxt = j + 1
                off_n = (wid * 32 + nxt) * _CHUNK
                @pl.when(jnp.logical_and(nxt < 32, nxt % 2 == 0))
                def _():
                    pltpu.make_async_copy(mat.at[pl.ds(off_n, _CHUNK)],
                                          buf.at[0], sems.at[0]).start()
                @pl.when(jnp.logical_and(nxt < 32, nxt % 2 == 1))
                def _():
                    pltpu.make_async_copy(mat.at[pl.ds(off_n, _CHUNK)],
                                          buf.at[1], sems.at[1]).start()
                off_j = (wid * 32 + j) * _CHUNK
                @pl.when(j % 2 == 0)
                def _():
                    pltpu.make_async_copy(mat.at[pl.ds(off_j, _CHUNK)],
                                          buf.at[0], sems.at[0]).wait()
                    process(0, _NVEC // _UNROLL)
                @pl.when(j % 2 == 1)
                def _():
                    pltpu.make_async_copy(mat.at[pl.ds(off_j, _CHUNK)],
                                          buf.at[1], sems.at[1]).wait()
                    process(1, _NVEC // _UNROLL)
                return c
            lax.fori_loop(0, 32, cb, 0)

        chunk_loop(shh)
        chunk_loop(sout)

        @pl.when(wid == 0)
        def _():
            pltpu.sync_copy(sbhh, buf.at[0, pl.ds(0, 2048)])
            process(0, 2048 // 16 // _UNROLL)

        @pl.when(wid == 1)
        def _():
            pltpu.sync_copy(sbout, buf.at[0, pl.ds(0, 2048)])
            process(0, 2048 // 16 // _UNROLL)

        pltpu.sync_copy(hist, out.at[pl.ds(wid * 16, 16)])

    scratch = [
        pltpu.VMEM((2, _CHUNK), jnp.float32),
        pltpu.VMEM((16, bins), jnp.int32),
        pltpu.SemaphoreType.DMA((2,)),
    ]
    if has_filter:
        scratch.append(pltpu.VMEM((128,), jnp.int32))

    return pl.kernel(
        body,
        mesh=mesh,
        out_type=jax.ShapeDtypeStruct((16 * _NW, bins), jnp.int32),
        scratch_types=scratch,
        compiler_params=pltpu.CompilerParams(needs_layout_passes=False),
    )


def _suffix_counts(hist_rows, bins):
    hist = jnp.sum(hist_rows.astype(jnp.float32), axis=0, keepdims=True)
    row = lax.broadcasted_iota(jnp.int32, (bins, bins), 0)
    col = lax.broadcasted_iota(jnp.int32, (bins, bins), 1)
    m = jnp.where(row >= col, 1.0, 0.0).astype(jnp.float32)
    counts_ge = lax.dot_general(hist, m, (((1,), (0,)), ((), ())),
                                precision=lax.Precision.HIGHEST,
                                preferred_element_type=jnp.float32)
    return counts_ge, hist


def _pick(counts_ge, hist, kcur, bins):
    ok = counts_ge >= kcur
    iota_b = lax.broadcasted_iota(jnp.int32, (1, bins), 1)
    bstar = jnp.max(jnp.where(ok, iota_b, -1))
    counts_gt = counts_ge - hist
    gt_at = jnp.sum(jnp.where(iota_b == bstar, counts_gt, 0.0))
    knext = kcur - gt_at
    return bstar, knext


def _make_sel(bins, level, k_total):
    def body(*refs):
        if level == 1:
            hist_ref, out_ref = refs
            kcur = jnp.float32(k_total)
            prev_prefix = jnp.int32(0)
        else:
            hist_ref, aux_ref, out_ref = refs
            kcur = aux_ref[1, 0].astype(jnp.float32)
            prev_prefix = aux_ref[0, 0]
        counts_ge, hist = _suffix_counts(hist_ref[...], bins)
        bstar, knext = _pick(counts_ge, hist, kcur, bins)
        prefix_next = prev_prefix * bins + bstar
        if level < 3:
            rowi = lax.broadcasted_iota(jnp.int32, (8, 128), 0)
            out_ref[...] = jnp.where(rowi == 0, prefix_next,
                                     knext.astype(jnp.int32))
        else:
            key = prefix_next.astype(jnp.uint32)
            keyi = lax.bitcast_convert_type(key, jnp.int32)
            bits = jnp.where(keyi < 0, key ^ jnp.uint32(0x80000000), ~key)
            thr = lax.bitcast_convert_type(bits, jnp.float32)
            out_ref[...] = jnp.full((8, 128), thr, jnp.float32)

    out_dtype = jnp.int32 if level < 3 else jnp.float32
    n_in = 1 if level == 1 else 2
    return pl.pallas_call(
        body,
        out_shape=jax.ShapeDtypeStruct((8, 128), out_dtype),
        in_specs=[pl.BlockSpec((512, bins), lambda: (0, 0))] +
                 ([pl.BlockSpec((8, 128), lambda: (0, 0))] if n_in == 2 else []),
        out_specs=pl.BlockSpec((8, 128), lambda: (0, 0)),
    )


_BM = 256
_BX = 512


def _mask_body(thr_ref, whh_ref, shh_ref, wout_ref, sout_ref,
               bhh_ref, sbhh_ref, bout_ref, sbout_ref,
               wmhh_ref, wmout_ref, bmhh_ref, bmout_ref):
    thr = thr_ref[0, 0]
    wmhh_ref[...] = jnp.where(shh_ref[...] >= thr, whh_ref[...], 0.0
                              ).astype(jnp.bfloat16)
    wmout_ref[...] = jnp.where(sout_ref[...] >= thr, wout_ref[...], 0.0
                               ).astype(jnp.bfloat16)
    bmhh_ref[...] = jnp.where(sbhh_ref[...] >= thr, bhh_ref[...], 0.0)
    bmout_ref[...] = jnp.where(sbout_ref[...] >= thr, bout_ref[...], 0.0)


def _apply_masks(thr, W_hh, s_hh, W_out, s_out, b_hh, s_b_hh, b_out, s_b_out):
    d_h, _ = W_hh.shape
    grid = (d_h // _BM,)
    row_spec = pl.BlockSpec((_BM, W_hh.shape[1]), lambda i: (i, 0))
    vec_spec = pl.BlockSpec((1, W_hh.shape[1]), lambda i: (0, 0))
    thr_spec = pl.BlockSpec((8, 128), lambda i: (0, 0))
    return pl.pallas_call(
        _mask_body,
        grid=grid,
        in_specs=[thr_spec, row_spec, row_spec, row_spec, row_spec,
                  vec_spec, vec_spec, vec_spec, vec_spec],
        out_specs=[row_spec, row_spec, vec_spec, vec_spec],
        out_shape=[
            jax.ShapeDtypeStruct(W_hh.shape, jnp.bfloat16),
            jax.ShapeDtypeStruct(W_out.shape, jnp.bfloat16),
            jax.ShapeDtypeStruct((1, d_h), jnp.float32),
            jax.ShapeDtypeStruct((1, d_h), jnp.float32),
        ],
    )(thr, W_hh, s_hh, W_out, s_out, b_hh, s_b_hh, b_out, s_b_out)


def _mm_body(x_ref, wih_ref, wmhh_ref, bmhh_ref, wmout_ref, bmout_ref,
             out_ref):
    nt = (((1,), (1,)), ((), ()))
    h = jnp.tanh(lax.dot_general(x_ref[...], wih_ref[...], nt,
                                 preferred_element_type=jnp.float32))
    h2 = jnp.tanh(lax.dot_general(h.astype(jnp.bfloat16), wmhh_ref[...], nt,
                                  preferred_element_type=jnp.float32)
                  + bmhh_ref[...])
    out_ref[...] = lax.dot_general(h2.astype(jnp.bfloat16), wmout_ref[...], nt,
                                   preferred_element_type=jnp.float32
                                   ) + bmout_ref[...]


def _forward(x_bf, wih_bf, wmhh, bmhh, wmout, bmout):
    n_tok, d_in = x_bf.shape
    d_h = wih_bf.shape[0]
    grid = (n_tok // _BX,)
    full = lambda s: pl.BlockSpec(s, lambda i: (0, 0))
    return pl.pallas_call(
        _mm_body,
        grid=grid,
        in_specs=[pl.BlockSpec((_BX, d_in), lambda i: (i, 0)),
                  full(wih_bf.shape), full(wmhh.shape), full((1, d_h)),
                  full(wmout.shape), full((1, d_h))],
        out_specs=pl.BlockSpec((_BX, d_h), lambda i: (i, 0)),
        out_shape=jax.ShapeDtypeStruct((n_tok, d_h), jnp.float32),
        compiler_params=pltpu.CompilerParams(
            vmem_limit_bytes=100 * 1024 * 1024),
    )(x_bf, wih_bf, wmhh, bmhh, wmout, bmout)


def kernel(x, W_ih, W_hh, b_hh, W_out, b_out, s_hh, s_b_hh, s_out, s_b_out):
    total = s_hh.size + s_b_hh.size + s_out.size + s_b_out.size
    k = max(1, int(_SPARSITY * total))

    shh_flat = s_hh.reshape(-1)
    sout_flat = s_out.reshape(-1)

    sc1 = _make_sc_hist(2048, 21, None)
    h1 = sc1(shh_flat, sout_flat, s_b_hh, s_b_out)
    p1 = _make_sel(2048, 1, k)(h1)

    sc2 = _make_sc_hist(2048, 10, 21)
    h2 = sc2(shh_flat, sout_flat, s_b_hh, s_b_out, p1)
    p2 = _make_sel(2048, 2, k)(h2, p1)

    sc3 = _make_sc_hist(1024, 0, 10)
    h3 = sc3(shh_flat, sout_flat, s_b_hh, s_b_out, p2)
    thr = _make_sel(1024, 3, k)(h3, p2)

    wmhh, wmout, bmhh, bmout = _apply_masks(
        thr, W_hh, s_hh, W_out, s_out,
        b_hh.reshape(1, -1), s_b_hh.reshape(1, -1),
        b_out.reshape(1, -1), s_b_out.reshape(1, -1))

    return _forward(x.astype(jnp.bfloat16), W_ih.astype(jnp.bfloat16),
                    wmhh, bmhh, wmout, bmout)


# speedup vs baseline: 16.7845x; 1.0941x over previous
import functools

import jax
import jax.numpy as jnp
from jax import lax
from jax.experimental import pallas as pl
from jax.experimental.pallas import tpu as pltpu
from jax.experimental.pallas import tpu_sc as plsc

_SPARSITY = 0.05


_NW = 32
_CHUNK = 4096
_NVEC = _CHUNK // 16


def _keyify(v):
    ui = lax.bitcast_convert_type(v, jnp.int32)
    sign = ui >> 31
    flip = lax.bitcast_convert_type(sign, jnp.uint32) | jnp.uint32(0x80000000)
    return lax.bitcast_convert_type(v, jnp.uint32) ^ flip


_UNROLL = 8


def _make_sc_hist(bins, shift, filt_shift):
    has_filter = filt_shift is not None
    rows = bins * 16 // 128
    mesh = plsc.VectorSubcoreMesh(core_axis_name="c", subcore_axis_name="s")

    def body(*refs):
        if has_filter:
            (shh, sout, sbhh, sbout, params, out,
             buf, hist, sems, prow) = refs
        else:
            (shh, sout, sbhh, sbout, out, buf, hist, sems) = refs
        wid = lax.axis_index("s") * 2 + lax.axis_index("c")
        lane = lax.iota(jnp.int32, 16)

        def clr(i, c):
            z = jnp.zeros((16,), jnp.int32)
            for l in range(8):
                hist[i, pl.ds(l * 16, 16)] = z
            return c
        lax.fori_loop(0, rows, clr, 0)

        if has_filter:
            pltpu.sync_copy(params.at[0], prow)
            prefix = prow[pl.ds(0, 16)].astype(jnp.uint32)

        def process(bslot, nvec8):
            def vb(i, c):
                for u in range(_UNROLL):
                    v = buf[bslot, pl.ds((i * _UNROLL + u) * 16, 16)]
                    key = _keyify(v)
                    field = ((key >> shift) & jnp.uint32(bins - 1)
                             ).astype(jnp.int32)
                    r = field >> 3
                    col = ((field & 7) << 4) + lane
                    if has_filter:
                        ok = (key >> filt_shift) == prefix
                        val = jnp.where(ok, 1, 0).astype(jnp.int32)
                    else:
                        val = jnp.full((16,), 1, jnp.int32)
                    plsc.addupdate_scatter(hist, [r, col], val)
                return c
            lax.fori_loop(0, nvec8, vb, 0)

        def chunk_loop(mat):
            first = mat.at[pl.ds(wid * 32 * _CHUNK, _CHUNK)]
            pltpu.make_async_copy(first, buf.at[0], sems.at[0]).start()

            def cb(j, c):
                nxt = j + 1
                off_n = (wid * 32 + nxt) * _CHUNK
                @pl.when(jnp.logical_and(nxt < 32, nxt % 2 == 0))
                def _():
                    pltpu.make_async_copy(mat.at[pl.ds(off_n, _CHUNK)],
                                          buf.at[0], sems.at[0]).start()
                @pl.when(jnp.logical_and(nxt < 32, nxt % 2 == 1))
                def _():
                    pltpu.make_async_copy(mat.at[pl.ds(off_n, _CHUNK)],
                                          buf.at[1], sems.at[1]).start()
                off_j = (wid * 32 + j) * _CHUNK
                @pl.when(j % 2 == 0)
                def _():
                    pltpu.make_async_copy(mat.at[pl.ds(off_j, _CHUNK)],
                                          buf.at[0], sems.at[0]).wait()
                    process(0, _NVEC // _UNROLL)
                @pl.when(j % 2 == 1)
                def _():
                    pltpu.make_async_copy(mat.at[pl.ds(off_j, _CHUNK)],
                                          buf.at[1], sems.at[1]).wait()
                    process(1, _NVEC // _UNROLL)
                return c
            lax.fori_loop(0, 32, cb, 0)

        chunk_loop(shh)
        chunk_loop(sout)

        @pl.when(wid == 0)
        def _():
            pltpu.sync_copy(sbhh, buf.at[0, pl.ds(0, 2048)])
            process(0, 2048 // 16 // _UNROLL)

        @pl.when(wid == 1)
        def _():
            pltpu.sync_copy(sbout, buf.at[0, pl.ds(0, 2048)])
            process(0, 2048 // 16 // _UNROLL)

        pltpu.sync_copy(hist, out.at[pl.ds(wid * rows, rows)])

    scratch = [
        pltpu.VMEM((2, _CHUNK), jnp.float32),
        pltpu.VMEM((rows, 128), jnp.int32),
        pltpu.SemaphoreType.DMA((2,)),
    ]
    if has_filter:
        scratch.append(pltpu.VMEM((128,), jnp.int32))

    return pl.kernel(
        body,
        mesh=mesh,
        out_type=jax.ShapeDtypeStruct((_NW * rows, 128), jnp.int32),
        scratch_types=scratch,
        compiler_params=pltpu.CompilerParams(needs_layout_passes=False),
    )


def _dot(a, b):
    return lax.dot_general(a, b, (((1,), (0,)), ((), ())),
                           precision=lax.Precision.HIGHEST,
                           preferred_element_type=jnp.float32)


def _suffix_counts(hist_raw, rows):
    f = hist_raw.astype(jnp.float32)
    acc = jnp.zeros((rows, 128), jnp.float32)
    for t in range(32):
        acc = acc + f[t * rows:(t + 1) * rows, :]
    c128 = lax.broadcasted_iota(jnp.int32, (128, 8), 0)
    g8 = lax.broadcasted_iota(jnp.int32, (128, 8), 1)
    r_mat = jnp.where((c128 >> 4) == g8, 1.0, 0.0)
    f4 = _dot(acc, r_mat)
    rowsum = jnp.sum(f4, axis=1, keepdims=True)
    ra = lax.broadcasted_iota(jnp.int32, (rows, rows), 0)
    rb = lax.broadcasted_iota(jnp.int32, (rows, rows), 1)
    s1 = _dot(jnp.where(rb > ra, 1.0, 0.0), rowsum)
    ga = lax.broadcasted_iota(jnp.int32, (8, 8), 0)
    gb = lax.broadcasted_iota(jnp.int32, (8, 8), 1)
    s2 = _dot(f4, jnp.where(ga >= gb, 1.0, 0.0))
    return s1 + s2, f4


def _pick(counts_ge, hist, kcur, rows):
    ok = counts_ge >= kcur
    iota_b = (lax.broadcasted_iota(jnp.int32, (rows, 8), 0) * 8
              + lax.broadcasted_iota(jnp.int32, (rows, 8), 1))
    bstar = jnp.max(jnp.where(ok, iota_b, -1))
    sel = iota_b == bstar
    gt_at = (jnp.sum(jnp.where(sel, counts_ge, 0.0))
             - jnp.sum(jnp.where(sel, hist, 0.0)))
    knext = kcur - gt_at
    return bstar, knext


def _make_sel(bins, level, k_total):
    rows = bins // 8

    def body(*refs):
        if level == 1:
            hist_ref, out_ref = refs
            kcur = jnp.float32(k_total)
            prev_prefix = jnp.int32(0)
        else:
            hist_ref, aux_ref, out_ref = refs
            kcur = aux_ref[1, 0].astype(jnp.float32)
            prev_prefix = aux_ref[0, 0]
        counts_ge, hist = _suffix_counts(hist_ref[...], rows)
        bstar, knext = _pick(counts_ge, hist, kcur, rows)
        prefix_next = prev_prefix * bins + bstar
        if level < 3:
            rowi = lax.broadcasted_iota(jnp.int32, (8, 128), 0)
            out_ref[...] = jnp.where(rowi == 0, prefix_next,
                                     knext.astype(jnp.int32))
        else:
            key = prefix_next.astype(jnp.uint32)
            keyi = lax.bitcast_convert_type(key, jnp.int32)
            bits = jnp.where(keyi < 0, key ^ jnp.uint32(0x80000000), ~key)
            thr = lax.bitcast_convert_type(bits, jnp.float32)
            out_ref[...] = jnp.full((8, 128), thr, jnp.float32)

    out_dtype = jnp.int32 if level < 3 else jnp.float32
    n_in = 1 if level == 1 else 2
    return pl.pallas_call(
        body,
        out_shape=jax.ShapeDtypeStruct((8, 128), out_dtype),
        in_specs=[pl.BlockSpec((32 * rows, 128), lambda: (0, 0))] +
                 ([pl.BlockSpec((8, 128), lambda: (0, 0))] if n_in == 2 else []),
        out_specs=pl.BlockSpec((8, 128), lambda: (0, 0)),
    )


_BM = 256
_BX = 512


def _mask_body(thr_ref, whh_ref, shh_ref, wout_ref, sout_ref,
               bhh_ref, sbhh_ref, bout_ref, sbout_ref,
               wmhh_ref, wmout_ref, bmhh_ref, bmout_ref):
    thr = thr_ref[0, 0]
    wmhh_ref[...] = jnp.where(shh_ref[...] >= thr, whh_ref[...], 0.0
                              ).astype(jnp.bfloat16)
    wmout_ref[...] = jnp.where(sout_ref[...] >= thr, wout_ref[...], 0.0
                               ).astype(jnp.bfloat16)
    bmhh_ref[...] = jnp.where(sbhh_ref[...] >= thr, bhh_ref[...], 0.0)
    bmout_ref[...] = jnp.where(sbout_ref[...] >= thr, bout_ref[...], 0.0)


def _apply_masks(thr, W_hh, s_hh, W_out, s_out, b_hh, s_b_hh, b_out, s_b_out):
    d_h, _ = W_hh.shape
    grid = (d_h // _BM,)
    row_spec = pl.BlockSpec((_BM, W_hh.shape[1]), lambda i: (i, 0))
    vec_spec = pl.BlockSpec((1, W_hh.shape[1]), lambda i: (0, 0))
    thr_spec = pl.BlockSpec((8, 128), lambda i: (0, 0))
    return pl.pallas_call(
        _mask_body,
        grid=grid,
        in_specs=[thr_spec, row_spec, row_spec, row_spec, row_spec,
                  vec_spec, vec_spec, vec_spec, vec_spec],
        out_specs=[row_spec, row_spec, vec_spec, vec_spec],
        out_shape=[
            jax.ShapeDtypeStruct(W_hh.shape, jnp.bfloat16),
            jax.ShapeDtypeStruct(W_out.shape, jnp.bfloat16),
            jax.ShapeDtypeStruct((1, d_h), jnp.float32),
            jax.ShapeDtypeStruct((1, d_h), jnp.float32),
        ],
    )(thr, W_hh, s_hh, W_out, s_out, b_hh, s_b_hh, b_out, s_b_out)


def _mm_body(x_ref, wih_ref, wmhh_ref, bmhh_ref, wmout_ref, bmout_ref,
             out_ref):
    nt = (((1,), (1,)), ((), ()))
    h = jnp.tanh(lax.dot_general(x_ref[...], wih_ref[...], nt,
                                 preferred_element_type=jnp.float32))
    h2 = jnp.tanh(lax.dot_general(h.astype(jnp.bfloat16), wmhh_ref[...], nt,
                                  preferred_element_type=jnp.float32)
                  + bmhh_ref[...])
    out_ref[...] = lax.dot_general(h2.astype(jnp.bfloat16), wmout_ref[...], nt,
                                   preferred_element_type=jnp.float32
                                   ) + bmout_ref[...]


def _forward(x_bf, wih_bf, wmhh, bmhh, wmout, bmout):
    n_tok, d_in = x_bf.shape
    d_h = wih_bf.shape[0]
    grid = (n_tok // _BX,)
    full = lambda s: pl.BlockSpec(s, lambda i: (0, 0))
    return pl.pallas_call(
        _mm_body,
        grid=grid,
        in_specs=[pl.BlockSpec((_BX, d_in), lambda i: (i, 0)),
                  full(wih_bf.shape), full(wmhh.shape), full((1, d_h)),
                  full(wmout.shape), full((1, d_h))],
        out_specs=pl.BlockSpec((_BX, d_h), lambda i: (i, 0)),
        out_shape=jax.ShapeDtypeStruct((n_tok, d_h), jnp.float32),
        compiler_params=pltpu.CompilerParams(
            vmem_limit_bytes=100 * 1024 * 1024),
    )(x_bf, wih_bf, wmhh, bmhh, wmout, bmout)


def kernel(x, W_ih, W_hh, b_hh, W_out, b_out, s_hh, s_b_hh, s_out, s_b_out):
    total = s_hh.size + s_b_hh.size + s_out.size + s_b_out.size
    k = max(1, int(_SPARSITY * total))

    shh_flat = s_hh.reshape(-1)
    sout_flat = s_out.reshape(-1)

    sc1 = _make_sc_hist(2048, 21, None)
    h1 = sc1(shh_flat, sout_flat, s_b_hh, s_b_out)
    p1 = _make_sel(2048, 1, k)(h1)

    sc2 = _make_sc_hist(2048, 10, 21)
    h2 = sc2(shh_flat, sout_flat, s_b_hh, s_b_out, p1)
    p2 = _make_sel(2048, 2, k)(h2, p1)

    sc3 = _make_sc_hist(1024, 0, 10)
    h3 = sc3(shh_flat, sout_flat, s_b_hh, s_b_out, p2)
    thr = _make_sel(1024, 3, k)(h3, p2)

    wmhh, wmout, bmhh, bmout = _apply_masks(
        thr, W_hh, s_hh, W_out, s_out,
        b_hh.reshape(1, -1), s_b_hh.reshape(1, -1),
        b_out.reshape(1, -1), s_b_out.reshape(1, -1))

    return _forward(x.astype(jnp.bfloat16), W_ih.astype(jnp.bfloat16),
                    wmhh, bmhh, wmout, bmout)
